# baseline (speedup 1.0000x reference)
import functools

import jax
import jax.numpy as jnp
from jax import lax
from jax.experimental import pallas as pl
from jax.experimental.pallas import tpu as pltpu
from jax.experimental.pallas import tpu_sc as plsc

_B, _S, _H = 4, 2048, 1024
_TOK = _B * _S
_NW = 32
_TPW = _TOK // _NW
_CHUNK = 32
_NCHUNK = _TPW // _CHUNK
_LANES = 16
_HV = _H // _LANES


def _make_sc_kernel():
    mesh = plsc.VectorSubcoreMesh(core_axis_name="c", subcore_axis_name="s")

    @functools.partial(
        pl.kernel,
        out_type=jax.ShapeDtypeStruct((_TOK, _H), jnp.float32),
        mesh=mesh,
        scratch_types=[
            pltpu.VMEM((_NCHUNK, _CHUNK), jnp.int32),
            pltpu.VMEM((_NCHUNK, _CHUNK), jnp.int32),
            pltpu.VMEM((_CHUNK, _H), jnp.float32),
            pltpu.VMEM((_CHUNK, _H), jnp.float32),
            pltpu.VMEM((_CHUNK, _H), jnp.float32),
            pltpu.SemaphoreType.DMA,
            pltpu.SemaphoreType.DMA,
        ],
    )
    def k(ids_hbm, tt_hbm, word_hbm, tttab_hbm, pos_hbm, out_hbm,
          idx_v, ttid_v, wbuf, pbuf, tbuf, sem_w, sem_t):
        wid = lax.axis_index("s") * 2 + lax.axis_index("c")
        base = wid * _TPW
        s0 = (wid % (_S // _TPW)) * _TPW
        pltpu.sync_copy(ids_hbm.at[wid], idx_v)
        pltpu.sync_copy(tt_hbm.at[wid], ttid_v)

        for c in range(_NCHUNK):
            gw = pltpu.async_copy(word_hbm.at[idx_v.at[c]], wbuf, sem_w)
            gt = pltpu.async_copy(tttab_hbm.at[ttid_v.at[c]], tbuf, sem_t)
            pltpu.sync_copy(pos_hbm.at[pl.ds(s0 + c * _CHUNK, _CHUNK)], pbuf)
            gw.wait()
            gt.wait()

            def tok_body(i, _):
                def h_body(hb, _):
                    for u in range(8):
                        h = (hb * 8 + u) * _LANES
                        pv = pbuf[i, pl.ds(h, _LANES)]
                        tv = tbuf[i, pl.ds(h, _LANES)]
                        plsc.addupdate(wbuf.at[i, pl.ds(h, _LANES)], pv + tv)
                    return 0

                lax.fori_loop(0, _HV // 8, h_body, 0)
                return 0

            lax.fori_loop(0, _CHUNK, tok_body, 0)
            pltpu.sync_copy(wbuf, out_hbm.at[pl.ds(base + c * _CHUNK, _CHUNK)])

    return k


_sc_embed = _make_sc_kernel()


def kernel(input_ids, token_type_ids, word_weight, token_type_weight, position_weight):
    ids = input_ids.astype(jnp.int32).reshape(_NW, _NCHUNK, _CHUNK)
    tt = token_type_ids.astype(jnp.int32).reshape(_NW, _NCHUNK, _CHUNK)
    out = _sc_embed(ids, tt, word_weight, token_type_weight, position_weight)
    return out.reshape(_B, _S, _H)


# speedup vs baseline: 1.9103x; 1.9103x over previous
import functools

import jax
import jax.numpy as jnp
from jax import lax
from jax.experimental import pallas as pl
from jax.experimental.pallas import tpu as pltpu
from jax.experimental.pallas import tpu_sc as plsc

_B, _S, _H = 4, 2048, 1024
_TOK = _B * _S
_NW = 32
_TPW = _TOK // _NW
_CHUNK = 32
_NCHUNK = _TPW // _CHUNK
_LANES = 16
_HV = _H // _LANES


def _make_sc_kernel():
    mesh = plsc.VectorSubcoreMesh(core_axis_name="c", subcore_axis_name="s")

    @functools.partial(
        pl.kernel,
        out_type=jax.ShapeDtypeStruct((_TOK, _H), jnp.float32),
        mesh=mesh,
        scratch_types=[
            pltpu.VMEM((_NCHUNK, _CHUNK), jnp.int32),
            pltpu.VMEM((_TPW * _LANES,), jnp.float32),
            pltpu.VMEM((2, _H), jnp.float32),
            pltpu.VMEM((_H,), jnp.float32),
            pltpu.VMEM((_CHUNK, _H), jnp.float32),
            pltpu.VMEM((_CHUNK, _H), jnp.float32),
            pltpu.VMEM((_CHUNK, _H), jnp.float32),
            pltpu.SemaphoreType.DMA,
            pltpu.SemaphoreType.DMA,
            pltpu.SemaphoreType.DMA,
            pltpu.SemaphoreType.DMA,
        ],
    )
    def k(ids_hbm, mexp_hbm, word_hbm, tttab_hbm, pos_hbm, out_hbm,
          idx_v, mexp_v, ttv, dv, wbuf0, wbuf1, pbuf,
          sem_g0, sem_g1, sem_o0, sem_o1):
        wbufs = (wbuf0, wbuf1)
        sem_g = (sem_g0, sem_g1)
        sem_o = (sem_o0, sem_o1)
        wid = lax.axis_index("s") * 2 + lax.axis_index("c")
        base = wid * _TPW
        s0 = (wid % (_S // _TPW)) * _TPW
        pltpu.sync_copy(ids_hbm.at[wid], idx_v)
        pltpu.sync_copy(mexp_hbm.at[pl.ds(base * _LANES, _TPW * _LANES)], mexp_v)
        pltpu.sync_copy(tttab_hbm, ttv)

        def d_body(hb, _):
            for u in range(4):
                h = (hb * 4 + u) * _LANES
                dv[pl.ds(h, _LANES)] = ttv[1, pl.ds(h, _LANES)] - ttv[0, pl.ds(h, _LANES)]
            return 0

        lax.fori_loop(0, _HV // 4, d_body, 0)

        scatters = [None, None]
        gathers = [pltpu.async_copy(word_hbm.at[idx_v.at[0]], wbufs[0], sem_g[0]), None]
        for c in range(_NCHUNK):
            nb = (c + 1) % 2
            if c + 1 < _NCHUNK:
                if scatters[nb] is not None:
                    scatters[nb].wait()
                    scatters[nb] = None
                gathers[nb] = pltpu.async_copy(
                    word_hbm.at[idx_v.at[c + 1]], wbufs[nb], sem_g[nb])
            pltpu.sync_copy(pos_hbm.at[pl.ds(s0 + c * _CHUNK, _CHUNK)], pbuf)
            gathers[c % 2].wait()
            wbuf = wbufs[c % 2]

            def h_body(hb, _, c=c, wbuf=wbuf):
                hoff = hb * _LANES
                t0h = ttv[0, pl.ds(hoff, _LANES)]
                dh = dv[pl.ds(hoff, _LANES)]

                def t_body(ib, _):
                    for u in range(4):
                        i = ib * 4 + u
                        m = mexp_v[pl.ds((c * _CHUNK + i) * _LANES, _LANES)]
                        pv = pbuf[i, pl.ds(hoff, _LANES)]
                        plsc.addupdate(wbuf.at[i, pl.ds(hoff, _LANES)],
                                       pv + t0h + m * dh)
                    return 0

                lax.fori_loop(0, _CHUNK // 4, t_body, 0)
                return 0

            lax.fori_loop(0, _HV, h_body, 0)
            scatters[c % 2] = pltpu.async_copy(
                wbuf, out_hbm.at[pl.ds(base + c * _CHUNK, _CHUNK)], sem_o[c % 2])
        for s in scatters:
            if s is not None:
                s.wait()

    return k


_sc_embed = _make_sc_kernel()


def kernel(input_ids, token_type_ids, word_weight, token_type_weight, position_weight):
    ids = input_ids.astype(jnp.int32).reshape(_NW, _NCHUNK, _CHUNK)
    mexp = jnp.broadcast_to(
        token_type_ids.astype(jnp.float32).reshape(_TOK, 1), (_TOK, _LANES)
    ).reshape(_TOK * _LANES)
    out = _sc_embed(ids, mexp, word_weight, token_type_weight, position_weight)
    return out.reshape(_B, _S, _H)


# speedup vs baseline: 3.2816x; 1.7179x over previous
import functools

import jax
import jax.numpy as jnp
from jax import lax
from jax.experimental import pallas as pl
from jax.experimental.pallas import tpu as pltpu
from jax.experimental.pallas import tpu_sc as plsc

_B, _S, _H = 4, 2048, 1024
_TOK = _B * _S
_NW = 32
_TPW = _TOK // _NW
_CHUNK = 32
_NCHUNK = _TPW // _CHUNK
_LANES = 16
_HV = _H // _LANES


def _make_sc_kernel():
    mesh = plsc.VectorSubcoreMesh(core_axis_name="c", subcore_axis_name="s")

    @functools.partial(
        pl.kernel,
        out_type=jax.ShapeDtypeStruct((_TOK, _H), jnp.float32),
        mesh=mesh,
        scratch_types=[
            pltpu.VMEM((_NCHUNK, _CHUNK), jnp.int32),
            pltpu.VMEM((_TPW * _LANES,), jnp.float32),
            pltpu.VMEM((2, _H), jnp.float32),
            pltpu.VMEM((_H,), jnp.float32),
            pltpu.VMEM((_CHUNK, _H), jnp.float32),
            pltpu.VMEM((_CHUNK, _H), jnp.float32),
            pltpu.VMEM((_CHUNK, _H), jnp.float32),
            pltpu.SemaphoreType.DMA,
            pltpu.SemaphoreType.DMA,
            pltpu.SemaphoreType.DMA,
            pltpu.SemaphoreType.DMA,
        ],
    )
    def k(ids_hbm, mexp_hbm, word_hbm, tttab_hbm, pos_hbm, out_hbm,
          idx_v, mexp_v, ttv, dv, wbuf0, wbuf1, pbuf,
          sem_g0, sem_g1, sem_o0, sem_o1):
        wbufs = (wbuf0, wbuf1)
        sem_g = (sem_g0, sem_g1)
        sem_o = (sem_o0, sem_o1)
        wid = lax.axis_index("s") * 2 + lax.axis_index("c")
        base = wid * _TPW
        s0 = (wid % (_S // _TPW)) * _TPW
        pltpu.sync_copy(ids_hbm.at[wid], idx_v)
        pltpu.sync_copy(mexp_hbm.at[pl.ds(base * _LANES, _TPW * _LANES)], mexp_v)
        pltpu.sync_copy(tttab_hbm, ttv)

        def d_body(hb, _):
            for u in range(4):
                h = (hb * 4 + u) * _LANES
                dv[pl.ds(h, _LANES)] = ttv[1, pl.ds(h, _LANES)] - ttv[0, pl.ds(h, _LANES)]
            return 0

        lax.fori_loop(0, _HV // 4, d_body, 0)

        scatters = [None, None]
        gathers = [pltpu.async_copy(word_hbm.at[idx_v.at[0]], wbufs[0], sem_g[0]), None]
        for c in range(_NCHUNK):
            nb = (c + 1) % 2
            if c + 1 < _NCHUNK:
                if scatters[nb] is not None:
                    scatters[nb].wait()
                    scatters[nb] = None
                gathers[nb] = pltpu.async_copy(
                    word_hbm.at[idx_v.at[c + 1]], wbufs[nb], sem_g[nb])
            pltpu.sync_copy(pos_hbm.at[pl.ds(s0 + c * _CHUNK, _CHUNK)], pbuf)
            gathers[c % 2].wait()
            wbuf = wbufs[c % 2]

            @plsc.parallel_loop(0, _HV, step=1)
            def h_body(hb, c=c, wbuf=wbuf):
                hoff = hb * _LANES
                t0h = ttv[0, pl.ds(hoff, _LANES)]
                dh = dv[pl.ds(hoff, _LANES)]

                @plsc.parallel_loop(0, _CHUNK, step=1, unroll=8)
                def t_body(i):
                    m = mexp_v[pl.ds((c * _CHUNK + i) * _LANES, _LANES)]
                    pv = pbuf[i, pl.ds(hoff, _LANES)]
                    plsc.addupdate(wbuf.at[i, pl.ds(hoff, _LANES)],
                                   pv + t0h + m * dh)
            scatters[c % 2] = pltpu.async_copy(
                wbuf, out_hbm.at[pl.ds(base + c * _CHUNK, _CHUNK)], sem_o[c % 2])
        for s in scatters:
            if s is not None:
                s.wait()

    return k


_sc_embed = _make_sc_kernel()


def kernel(input_ids, token_type_ids, word_weight, token_type_weight, position_weight):
    ids = input_ids.astype(jnp.int32).reshape(_NW, _NCHUNK, _CHUNK)
    mexp = jnp.broadcast_to(
        token_type_ids.astype(jnp.float32).reshape(_TOK, 1), (_TOK, _LANES)
    ).reshape(_TOK * _LANES)
    out = _sc_embed(ids, mexp, word_weight, token_type_weight, position_weight)
    return out.reshape(_B, _S, _H)


# speedup vs baseline: 4.3570x; 1.3277x over previous
import functools

import jax
import jax.numpy as jnp
from jax import lax
from jax.experimental import pallas as pl
from jax.experimental.pallas import tpu as pltpu
from jax.experimental.pallas import tpu_sc as plsc

_B, _S, _H = 4, 2048, 1024
_TOK = _B * _S
_NW = 32
_TPW = _TOK // _NW
_CHUNK = 16
_NCHUNK = _TPW // _CHUNK
_LANES = 16
_HV = _H // _LANES
_NWB = 4
_NPB = 3


def _make_sc_kernel():
    mesh = plsc.VectorSubcoreMesh(core_axis_name="c", subcore_axis_name="s")

    @functools.partial(
        pl.kernel,
        out_type=jax.ShapeDtypeStruct((_TOK, _H), jnp.float32),
        mesh=mesh,
        scratch_types=(
            [pltpu.VMEM((_NCHUNK, _CHUNK), jnp.int32),
             pltpu.VMEM((_TPW * _LANES,), jnp.float32),
             pltpu.VMEM((2, _H), jnp.float32),
             pltpu.VMEM((_H,), jnp.float32)]
            + [pltpu.VMEM((_CHUNK, _H), jnp.float32)] * (_NWB + _NPB)
            + [pltpu.SemaphoreType.DMA] * (2 * _NWB + _NPB)
        ),
    )
    def k(ids_hbm, mexp_hbm, word_hbm, tttab_hbm, pos_hbm, out_hbm,
          idx_v, mexp_v, ttv, dv, *bufs_and_sems):
        wbufs = bufs_and_sems[:_NWB]
        pbufs = bufs_and_sems[_NWB:_NWB + _NPB]
        sem_g = bufs_and_sems[_NWB + _NPB:2 * _NWB + _NPB]
        sem_o = bufs_and_sems[2 * _NWB + _NPB:3 * _NWB + _NPB]
        sem_p = bufs_and_sems[3 * _NWB + _NPB:]
        wid = lax.axis_index("s") * 2 + lax.axis_index("c")
        base = wid * _TPW
        s0 = (wid % (_S // _TPW)) * _TPW
        pltpu.sync_copy(ids_hbm.at[wid], idx_v)
        pltpu.sync_copy(mexp_hbm.at[pl.ds(base * _LANES, _TPW * _LANES)], mexp_v)
        pltpu.sync_copy(tttab_hbm, ttv)

        def d_body(hb, _):
            for u in range(4):
                h = (hb * 4 + u) * _LANES
                dv[pl.ds(h, _LANES)] = ttv[1, pl.ds(h, _LANES)] - ttv[0, pl.ds(h, _LANES)]
            return 0

        lax.fori_loop(0, _HV // 4, d_body, 0)

        def gather(c):
            return pltpu.async_copy(
                word_hbm.at[idx_v.at[c]], wbufs[c % _NWB], sem_g[c % _NWB])

        def posdma(c):
            return pltpu.async_copy(
                pos_hbm.at[pl.ds(s0 + c * _CHUNK, _CHUNK)],
                pbufs[c % _NPB], sem_p[c % _NPB])

        gathers = [None] * _NWB
        posdmas = [None] * _NPB
        scatters = [None] * _NWB
        for c in range(2):
            gathers[c % _NWB] = gather(c)
            posdmas[c % _NPB] = posdma(c)

        for c in range(_NCHUNK):
            if c + 2 < _NCHUNK:
                bi = (c + 2) % _NWB
                if scatters[bi] is not None:
                    scatters[bi].wait()
                    scatters[bi] = None
                gathers[bi] = gather(c + 2)
                posdmas[(c + 2) % _NPB] = posdma(c + 2)
            gathers[c % _NWB].wait()
            posdmas[c % _NPB].wait()
            wbuf = wbufs[c % _NWB]
            pbuf = pbufs[c % _NPB]

            @plsc.parallel_loop(0, _HV, step=1)
            def h_body(hb, c=c, wbuf=wbuf, pbuf=pbuf):
                hoff = hb * _LANES
                t0h = ttv[0, pl.ds(hoff, _LANES)]
                dh = dv[pl.ds(hoff, _LANES)]

                @plsc.parallel_loop(0, _CHUNK, step=1, unroll=8)
                def t_body(i):
                    m = mexp_v[pl.ds((c * _CHUNK + i) * _LANES, _LANES)]
                    pv = pbuf[i, pl.ds(hoff, _LANES)]
                    plsc.addupdate(wbuf.at[i, pl.ds(hoff, _LANES)],
                                   pv + t0h + m * dh)

            scatters[c % _NWB] = pltpu.async_copy(
                wbuf, out_hbm.at[pl.ds(base + c * _CHUNK, _CHUNK)], sem_o[c % _NWB])
        for s in scatters:
            if s is not None:
                s.wait()

    return k


_sc_embed = _make_sc_kernel()


def kernel(input_ids, token_type_ids, word_weight, token_type_weight, position_weight):
    ids = input_ids.astype(jnp.int32).reshape(_NW, _NCHUNK, _CHUNK)
    mexp = jnp.broadcast_to(
        token_type_ids.astype(jnp.float32).reshape(_TOK, 1), (_TOK, _LANES)
    ).reshape(_TOK * _LANES)
    out = _sc_embed(ids, mexp, word_weight, token_type_weight, position_weight)
    return out.reshape(_B, _S, _H)


# speedup vs baseline: 4.4115x; 1.0125x over previous
import functools

import jax
import jax.numpy as jnp
from jax import lax
from jax.experimental import pallas as pl
from jax.experimental.pallas import tpu as pltpu
from jax.experimental.pallas import tpu_sc as plsc

_B, _S, _H = 4, 2048, 1024
_TOK = _B * _S
_NW = 32
_TPW = _TOK // _NW
_CHUNK = 16
_NJ = 4
_NCHUNK = _TPW // _CHUNK
_SPW = _S // _NW
_LANES = 16
_HV = _H // _LANES
_NWB = 5
_PF = 3
_NPB = 2


def _make_sc_kernel():
    mesh = plsc.VectorSubcoreMesh(core_axis_name="c", subcore_axis_name="s")

    @functools.partial(
        pl.kernel,
        out_type=jax.ShapeDtypeStruct((_TOK, _H), jnp.float32),
        mesh=mesh,
        scratch_types=(
            [pltpu.VMEM((_NCHUNK, _CHUNK), jnp.int32),
             pltpu.VMEM((_TPW * _LANES,), jnp.float32),
             pltpu.VMEM((2, _H), jnp.float32),
             pltpu.VMEM((_H,), jnp.float32)]
            + [pltpu.VMEM((_CHUNK, _H), jnp.float32)] * (_NWB + _NPB)
            + [pltpu.SemaphoreType.DMA] * (2 * _NWB + _NPB)
        ),
    )
    def k(ids_hbm, mexp_hbm, word_hbm, tttab_hbm, pos_hbm, out_hbm,
          idx_v, mexp_v, ttv, dv, *bufs_and_sems):
        wbufs = bufs_and_sems[:_NWB]
        pbufs = bufs_and_sems[_NWB:_NWB + _NPB]
        sem_g = bufs_and_sems[_NWB + _NPB:2 * _NWB + _NPB]
        sem_o = bufs_and_sems[2 * _NWB + _NPB:3 * _NWB + _NPB]
        sem_p = bufs_and_sems[3 * _NWB + _NPB:]
        wid = lax.axis_index("s") * 2 + lax.axis_index("c")
        s0 = wid * _SPW
        pltpu.sync_copy(ids_hbm.at[wid], idx_v)
        pltpu.sync_copy(mexp_hbm.at[wid], mexp_v)
        pltpu.sync_copy(tttab_hbm, ttv)

        def d_body(hb, _):
            for u in range(4):
                h = (hb * 4 + u) * _LANES
                dv[pl.ds(h, _LANES)] = ttv[1, pl.ds(h, _LANES)] - ttv[0, pl.ds(h, _LANES)]
            return 0

        lax.fori_loop(0, _HV // 4, d_body, 0)

        def gather(c):
            return pltpu.async_copy(
                word_hbm.at[idx_v.at[c]], wbufs[c % _NWB], sem_g[c % _NWB])

        def posdma(j):
            return pltpu.async_copy(
                pos_hbm.at[pl.ds(s0 + j * _CHUNK, _CHUNK)],
                pbufs[j % _NPB], sem_p[j % _NPB])

        gathers = [None] * _NWB
        posdmas = [None] * _NPB
        scatters = [None] * _NWB
        for c in range(_PF):
            gathers[c % _NWB] = gather(c)
        for j in range(_NPB):
            posdmas[j % _NPB] = posdma(j)

        for c in range(_NCHUNK):
            j, b = divmod(c, _B)
            if c + _PF < _NCHUNK:
                bi = (c + _PF) % _NWB
                if scatters[bi] is not None:
                    scatters[bi].wait()
                    scatters[bi] = None
                gathers[bi] = gather(c + _PF)
            if b == 0:
                if _NPB - 1 <= j < _NJ - 1:
                    posdmas[(j + 1) % _NPB] = posdma(j + 1)
                posdmas[j % _NPB].wait()
            gathers[c % _NWB].wait()
            wbuf = wbufs[c % _NWB]
            pbuf = pbufs[j % _NPB]

            @plsc.parallel_loop(0, _HV, step=1)
            def h_body(hb, c=c, wbuf=wbuf, pbuf=pbuf):
                hoff = hb * _LANES
                t0h = ttv[0, pl.ds(hoff, _LANES)]
                dh = dv[pl.ds(hoff, _LANES)]

                @plsc.parallel_loop(0, _CHUNK, step=1, unroll=8)
                def t_body(i):
                    m = mexp_v[pl.ds((c * _CHUNK + i) * _LANES, _LANES)]
                    pv = pbuf[i, pl.ds(hoff, _LANES)]
                    plsc.addupdate(wbuf.at[i, pl.ds(hoff, _LANES)],
                                   pv + t0h + m * dh)

            scatters[c % _NWB] = pltpu.async_copy(
                wbuf,
                out_hbm.at[pl.ds(b * _S + s0 + j * _CHUNK, _CHUNK)],
                sem_o[c % _NWB])
        for s in scatters:
            if s is not None:
                s.wait()

    return k


_sc_embed = _make_sc_kernel()


def kernel(input_ids, token_type_ids, word_weight, token_type_weight, position_weight):
    ids4 = input_ids.astype(jnp.int32).reshape(_B, _NW, _NJ, _CHUNK)
    ids = jnp.transpose(ids4, (1, 2, 0, 3)).reshape(_NW, _NCHUNK, _CHUNK)
    tt4 = token_type_ids.astype(jnp.float32).reshape(_B, _NW, _NJ, _CHUNK)
    mexp = jnp.broadcast_to(
        jnp.transpose(tt4, (1, 2, 0, 3))[..., None],
        (_NW, _NJ, _B, _CHUNK, _LANES),
    ).reshape(_NW, _NCHUNK * _CHUNK * _LANES)
    out = _sc_embed(ids, mexp, word_weight, token_type_weight, position_weight)
    return out.reshape(_B, _S, _H)
